# baseline (speedup 1.0000x reference)
import functools
import jax
import jax.numpy as jnp
from jax import lax
from jax.experimental import pallas as pl
from jax.experimental.pallas import tpu as pltpu
from jax.experimental.pallas import tpu_sc as plsc

_NC = 2
_NS = 16
_NW = _NC * _NS
_B = 16384
_D = 64
_D2 = 128
_BPW = _B // _NW
_CHUNK = 128
_NCH = _BPW // _CHUNK
_IDXROWS = _B // _CHUNK
_BLK = 2048


def _sc_gather_body(uidx_hbm, iidx_hbm, utab_hbm, itab_hbm, out_hbm,
                    uidx_v, iidx_v, urows_v, irows_v, sem):
  wid = lax.axis_index("s") * _NC + lax.axis_index("c")
  base = wid * _BPW
  row0 = wid * _NCH
  pltpu.sync_copy(uidx_hbm.at[pl.ds(row0, _NCH)], uidx_v)
  pltpu.sync_copy(iidx_hbm.at[pl.ds(row0, _NCH)], iidx_v)

  for ch in range(_NCH):
    cu = pltpu.async_copy(utab_hbm.at[uidx_v.at[ch]], urows_v, sem)
    ci = pltpu.async_copy(itab_hbm.at[iidx_v.at[ch]], irows_v, sem)
    cu.wait()
    ci.wait()
    dst = pl.ds(base + ch * _CHUNK, _CHUNK)
    pltpu.sync_copy(urows_v, out_hbm.at[0, dst])
    pltpu.sync_copy(irows_v, out_hbm.at[1, dst])


def _sc_gather(uidx2, iidx2, utab, itab):
  mesh = plsc.VectorSubcoreMesh(core_axis_name="c", subcore_axis_name="s")
  k = functools.partial(
      pl.kernel, mesh=mesh,
      compiler_params=pltpu.CompilerParams(use_tc_tiling_on_sc=True),
      out_type=jax.ShapeDtypeStruct((2, _B, _D2), jnp.float32),
      scratch_types=[
          pltpu.VMEM((_NCH, _CHUNK), jnp.int32),
          pltpu.VMEM((_NCH, _CHUNK), jnp.int32),
          pltpu.VMEM((_CHUNK, _D2), jnp.float32),
          pltpu.VMEM((_CHUNK, _D2), jnp.float32),
          pltpu.SemaphoreType.DMA,
      ],
  )(_sc_gather_body)
  return k(uidx2, iidx2, utab, itab)


def _mlp_body(x_ref, up_ref, ip_ref, w1u_ref, w1v_ref, b1_ref, w2_ref, b2_ref,
              w3_ref, b3_ref, w4_ref, b4_ref, out_ref):
  u2 = x_ref[0]
  v2 = x_ref[1]
  u = jnp.where(up_ref[...] > 0, u2[:, _D:], u2[:, :_D])
  v = jnp.where(ip_ref[...] > 0, v2[:, _D:], v2[:, :_D])
  h = jnp.dot(u, w1u_ref[...], preferred_element_type=jnp.float32)
  h = h + jnp.dot(v, w1v_ref[...], preferred_element_type=jnp.float32)
  h = jnp.maximum(h + b1_ref[...], 0.0)
  h = jnp.dot(h, w2_ref[...], preferred_element_type=jnp.float32)
  h = jnp.maximum(h + b2_ref[...], 0.0)
  h = jnp.dot(h, w3_ref[...], preferred_element_type=jnp.float32)
  h = jnp.maximum(h + b3_ref[...], 0.0)
  out_ref[...] = (jnp.dot(h, w4_ref[...], preferred_element_type=jnp.float32)
                  + b4_ref[...])


def _tc_mlp(x, upar, ipar, w1u, w1v, b1, w2, b2, w3, b3, w4, b4):
  full = lambda shape: pl.BlockSpec(shape, lambda i: tuple(0 for _ in shape))
  return pl.pallas_call(
      _mlp_body,
      grid=(_B // _BLK,),
      in_specs=[
          pl.BlockSpec((2, _BLK, _D2), lambda i: (0, i, 0)),
          pl.BlockSpec((_BLK, 1), lambda i: (i, 0)),
          pl.BlockSpec((_BLK, 1), lambda i: (i, 0)),
          full((_D, 32)), full((_D, 32)), full((1, 32)),
          full((32, 16)), full((1, 16)),
          full((16, 8)), full((1, 8)),
          full((8, 1)), full((1, 1)),
      ],
      out_specs=pl.BlockSpec((_BLK, 1), lambda i: (i, 0)),
      out_shape=jax.ShapeDtypeStruct((_B, 1), jnp.float32),
  )(x, upar, ipar, w1u, w1v, b1, w2, b2, w3, b3, w4, b4)


@jax.jit
def kernel(user, items, user_table, item_table, W1, b1, W2, b2, W3, b3, W4, b4):
  u32 = user.astype(jnp.int32)
  i32 = items.astype(jnp.int32)
  uh = (u32 >> 1).reshape(_IDXROWS, _CHUNK)
  ih = (i32 >> 1).reshape(_IDXROWS, _CHUNK)
  upar = (u32 & 1).astype(jnp.float32).reshape(_B, 1)
  ipar = (i32 & 1).astype(jnp.float32).reshape(_B, 1)
  ut2 = user_table.reshape(-1, _D2)
  it2 = item_table.reshape(-1, _D2)
  x = _sc_gather(uh, ih, ut2, it2)
  return _tc_mlp(x, upar, ipar,
                 W1[:, :_D].T, W1[:, _D:].T, b1.reshape(1, 32),
                 W2.T, b2.reshape(1, 16), W3.T, b3.reshape(1, 8),
                 W4.T, b4.reshape(1, 1))
